# baseline (speedup 1.0000x reference)
import jax
import jax.numpy as jnp
from jax import lax
from jax.experimental import pallas as pl

_K = 512
_D = 64
_BB = 4


def _vq_body(z_ref, w_ref, wt_ref, out_ref):
    w = w_ref[...]
    wt = wt_ref[...]
    wn = jnp.sum(w * w, axis=1, keepdims=True)
    H, W = z_ref.shape[2], z_ref.shape[3]
    hw = H * W
    for j in range(_BB):
        x = z_ref[j].reshape(_D, hw)
        zn = jnp.sum(x * x, axis=0, keepdims=True)
        scores = lax.dot_general(
            w, x, dimension_numbers=(((1,), (0,)), ((), ())),
            preferred_element_type=jnp.float32)
        d = (zn + wn) - 2.0 * scores

        m = jnp.min(d, axis=0, keepdims=True)
        rows = lax.broadcasted_iota(jnp.int32, (_K, hw), 0)
        idx = jnp.min(jnp.where(d <= m, rows, _K), axis=0)

        onehot = (rows == idx[None, :]).astype(jnp.float32)
        res = lax.dot_general(
            wt, onehot, dimension_numbers=(((1,), (0,)), ((), ())),
            preferred_element_type=jnp.float32)
        out_ref[j] = res.reshape(_D, H, W)


def kernel(z_e, weight):
    B, C, H, W = z_e.shape
    wt = jnp.transpose(weight, (1, 0))

    return pl.pallas_call(
        _vq_body,
        grid=(B // _BB,),
        in_specs=[
            pl.BlockSpec((_BB, C, H, W), lambda b: (b, 0, 0, 0)),
            pl.BlockSpec((_K, _D), lambda b: (0, 0)),
            pl.BlockSpec((_D, _K), lambda b: (0, 0)),
        ],
        out_specs=pl.BlockSpec((_BB, C, H, W), lambda b: (b, 0, 0, 0)),
        out_shape=jax.ShapeDtypeStruct((B, C, H, W), jnp.float32),
    )(z_e, weight, wt)


# speedup vs baseline: 1.5519x; 1.5519x over previous
import functools

import jax
import jax.numpy as jnp
from jax import lax
from jax.experimental import pallas as pl
from jax.experimental.pallas import tpu as pltpu
from jax.experimental.pallas import tpu_sc as plsc

_K = 512
_D = 64
_NC = 2
_NS = 16
_NW = _NC * _NS
_L = 16
_BB = 8


def _argmin_body(z_ref, w_ref, idx_ref):
    w = w_ref[...]
    wn = jnp.sum(w * w, axis=1, keepdims=True)
    hw = z_ref.shape[2]
    rows = lax.broadcasted_iota(jnp.int32, (_K, hw), 0)
    for j in range(_BB):
        x = z_ref[j]
        zn = jnp.sum(x * x, axis=0, keepdims=True)
        scores = lax.dot_general(
            w, x, dimension_numbers=(((1,), (0,)), ((), ())),
            preferred_element_type=jnp.float32)
        d = (zn + wn) - 2.0 * scores
        m = jnp.min(d, axis=0, keepdims=True)
        idx = jnp.min(jnp.where(d <= m, rows, _K),
                      axis=0, keepdims=True)
        idx_ref[0, pl.ds(j, 1), :] = idx


def _sc_gather_body(wt_hbm, idx_hbm, out_hbm, wt_v, idx_v, out_v):
    wid = lax.axis_index("s") * _NC + lax.axis_index("c")
    per_w = idx_hbm.shape[0] // _NW
    hw = idx_hbm.shape[1]

    pltpu.sync_copy(wt_hbm, wt_v)
    for j in range(per_w):
        b = wid * per_w + j
        pltpu.sync_copy(idx_hbm.at[b], idx_v)

        @plsc.parallel_loop(0, hw // _L, unroll=2)
        def grp(g):
            vidx = idx_v[pl.ds(g * _L, _L)]
            for c in range(_D):
                row = plsc.load_gather(
                    wt_v, [jnp.full((_L,), c, jnp.int32), vidx])
                out_v[c, pl.ds(g * _L, _L)] = row

        pltpu.sync_copy(out_v, out_hbm.at[b])


def kernel(z_e, weight):
    B, C, H, W = z_e.shape
    hw = H * W
    z = z_e.reshape(B, C, hw)
    wt = jnp.transpose(weight, (1, 0))

    idx = pl.pallas_call(
        _argmin_body,
        grid=(B // _BB,),
        in_specs=[
            pl.BlockSpec((_BB, C, hw), lambda b: (b, 0, 0)),
            pl.BlockSpec((_K, _D), lambda b: (0, 0)),
        ],
        out_specs=pl.BlockSpec((1, _BB, hw), lambda b: (b, 0, 0)),
        out_shape=jax.ShapeDtypeStruct((B // _BB, _BB, hw), jnp.int32),
    )(z, weight)

    sc_gather = functools.partial(
        pl.kernel,
        out_type=jax.ShapeDtypeStruct((B, _D, hw), jnp.float32),
        mesh=plsc.VectorSubcoreMesh(core_axis_name="c", subcore_axis_name="s"),
        scratch_types=[
            pltpu.VMEM((_D, _K), jnp.float32),
            pltpu.VMEM((hw,), jnp.int32),
            pltpu.VMEM((_D, hw), jnp.float32),
        ],
        compiler_params=pltpu.CompilerParams(needs_layout_passes=False),
    )(_sc_gather_body)

    zq = sc_gather(wt, idx.reshape(B, hw))
    return zq.reshape(B, C, H, W)
